# baseline (speedup 1.0000x reference)
import functools

import jax
import jax.numpy as jnp
from jax import lax
from jax.experimental import pallas as pl
from jax.experimental.pallas import tpu as pltpu
from jax.experimental.pallas import tpu_sc as plsc

_V = 1000
_H = 128
_NC = 2
_NS = 16
_NW = _NC * _NS


def _table_body(e_ref, w_ref, b_ref, t_ref):
    t_ref[...] = (
        jnp.dot(e_ref[...], w_ref[...].T, preferred_element_type=jnp.float32)
        + b_ref[...]
    )


def _make_table(embed, W, b2d):
    return pl.pallas_call(
        _table_body,
        out_shape=jax.ShapeDtypeStruct((_V, _V), jnp.float32),
    )(embed, W, b2d)


def _make_gather(n_rows):
    rows_per_w = n_rows // _NW
    chunk = 32
    n_chunks = rows_per_w // chunk
    mesh = plsc.VectorSubcoreMesh(core_axis_name="c", subcore_axis_name="s")

    @functools.partial(
        pl.kernel,
        mesh=mesh,
        compiler_params=pltpu.CompilerParams(use_tc_tiling_on_sc=False),
        out_type=jax.ShapeDtypeStruct((n_rows, _V), jnp.float32),
        scratch_types=[
            pltpu.VMEM_SHARED((_V, _V), jnp.float32),
            pltpu.VMEM((rows_per_w,), jnp.int32),
            pltpu.VMEM((chunk, _V), jnp.float32),
            pltpu.VMEM((chunk, _V), jnp.float32),
            pltpu.SemaphoreType.DMA,
            pltpu.SemaphoreType.DMA,
            pltpu.SemaphoreType.DMA,
            pltpu.SemaphoreType.DMA,
        ],
    )
    def gather(table_hbm, idx_hbm, out_hbm, table_sh, idx_v,
               r0, r1, gs0, gs1, ws0, ws1):
        sid = lax.axis_index("s")
        wid = sid * _NC + lax.axis_index("c")
        base = wid * rows_per_w

        @pl.when(sid == 0)
        def _():
            pltpu.sync_copy(table_hbm, table_sh)

        pltpu.sync_copy(idx_hbm.at[pl.ds(base, rows_per_w)], idx_v)
        plsc.subcore_barrier()

        bufs = ((r0, gs0, ws0), (r1, gs1, ws1))

        def start_gather(ci, r, gs):
            pltpu.async_copy(
                table_sh.at[idx_v.at[pl.ds(ci * chunk, chunk)]], r, gs
            )

        for b in range(2):
            start_gather(b, bufs[b][0], bufs[b][1])

        def body(i, carry):
            g = i * 2
            for b in range(2):
                ci = g + b
                r, gs, ws = bufs[b]
                pltpu.make_async_copy(
                    table_sh.at[idx_v.at[pl.ds(ci * chunk, chunk)]], r, gs
                ).wait()
                pltpu.async_copy(r, out_hbm.at[pl.ds(base + ci * chunk, chunk)], ws)
            for b in range(2):
                ci = g + b
                r, gs, ws = bufs[b]

                @pl.when(ci + 2 < n_chunks)
                def _():
                    pltpu.make_async_copy(
                        r, out_hbm.at[pl.ds(base, chunk)], ws
                    ).wait()
                    start_gather(ci + 2, r, gs)

            return carry

        lax.fori_loop(0, n_chunks // 2, body, 0)

        for b in range(2):
            r, gs, ws = bufs[b]
            pltpu.make_async_copy(r, out_hbm.at[pl.ds(base, chunk)], ws).wait()

    return gather


def kernel(input_ids, embed, W, b):
    bsz, seq = input_ids.shape
    table = _make_table(embed, W, b.reshape(1, _V))
    ids = input_ids.reshape(-1).astype(jnp.int32)
    out = _make_gather(bsz * seq)(table, ids)
    return out.reshape(bsz, seq, _V)
